# Initial kernel scaffold; baseline (speedup 1.0000x reference)
import jax
import jax.numpy as jnp
from jax.experimental import pallas as pl


def kernel(x, weights):
    raise NotImplementedError("write your pallas kernel here")


# speedup vs baseline: 1.6836x; 1.6836x over previous
import functools

import jax
import jax.numpy as jnp
from jax import lax
from jax.experimental import pallas as pl
from jax.experimental.pallas import tpu as pltpu
from jax.experimental.pallas import tpu_sc as plsc

_D = 64
_NC = 2
_NS = 16
_NW = _NC * _NS
_CHUNK = 128


def _emb_lookup(x_flat, weights, b_per_w, nchunk):
    total = _NW * b_per_w
    mesh = plsc.VectorSubcoreMesh(core_axis_name="c", subcore_axis_name="s")

    @functools.partial(
        pl.kernel,
        out_type=jax.ShapeDtypeStruct((total, _D), jnp.float32),
        mesh=mesh,
        scratch_types=[
            pltpu.VMEM((nchunk, _CHUNK), jnp.int32),
            pltpu.VMEM((_CHUNK, _D), jnp.float32),
            pltpu.SemaphoreType.DMA,
        ],
        compiler_params=pltpu.CompilerParams(use_tc_tiling_on_sc=False),
    )
    def emb(x_hbm, tbl_hbm, out_hbm, idx_v, rows_v, sem):
        wid = lax.axis_index("s") * _NC + lax.axis_index("c")
        base = wid * b_per_w
        pltpu.sync_copy(x_hbm.at[wid], idx_v)

        def chunk_body(c, carry):
            pltpu.async_copy(tbl_hbm.at[idx_v.at[c]], rows_v, sem).wait()
            pltpu.sync_copy(
                rows_v, out_hbm.at[pl.ds(base + c * _CHUNK, _CHUNK)]
            )
            return carry

        lax.fori_loop(0, nchunk, chunk_body, 0)

    return emb(x_flat, weights)


def kernel(x, weights):
    batch = x.size
    b_per_w = batch // _NW
    nchunk = b_per_w // _CHUNK
    x_flat = x.reshape(_NW, nchunk, _CHUNK)
    out = _emb_lookup(x_flat, weights, b_per_w, nchunk)
    return out.reshape(x.shape + (_D,))


# speedup vs baseline: 1.8731x; 1.1126x over previous
import functools

import jax
import jax.numpy as jnp
from jax import lax
from jax.experimental import pallas as pl
from jax.experimental.pallas import tpu as pltpu
from jax.experimental.pallas import tpu_sc as plsc

_D = 64
_NC = 2
_NS = 16
_NW = _NC * _NS
_CHUNK = 128
_NBUF = 8


def _emb_lookup(x_flat, weights, b_per_w, nchunk):
    total = _NW * b_per_w
    nouter = nchunk // _NBUF
    mesh = plsc.VectorSubcoreMesh(core_axis_name="c", subcore_axis_name="s")

    @functools.partial(
        pl.kernel,
        out_type=jax.ShapeDtypeStruct((total, _D), jnp.float32),
        mesh=mesh,
        scratch_types=(
            [pltpu.VMEM((nchunk, _CHUNK), jnp.int32)]
            + [pltpu.VMEM((_NBUF, _CHUNK, _D), jnp.float32)]
            + [pltpu.SemaphoreType.DMA] * (2 * _NBUF)
        ),
        compiler_params=pltpu.CompilerParams(use_tc_tiling_on_sc=False),
    )
    def emb(x_hbm, tbl_hbm, out_hbm, idx_v, rows_v, *sems):
        gsem = sems[:_NBUF]
        wsem = sems[_NBUF:]
        wid = lax.axis_index("s") * _NC + lax.axis_index("c")
        base = wid * b_per_w
        pltpu.sync_copy(x_hbm.at[wid], idx_v)

        def gather_descr(c, b):
            return pltpu.make_async_copy(
                tbl_hbm.at[idx_v.at[c]], rows_v.at[b], gsem[b]
            )

        def write_descr(c, b):
            return pltpu.make_async_copy(
                rows_v.at[b], out_hbm.at[pl.ds(base + c * _CHUNK, _CHUNK)],
                wsem[b],
            )

        for b in range(_NBUF):
            gather_descr(b, b).start()

        def outer(o, carry):
            c0 = o * _NBUF
            for b in range(_NBUF):
                gather_descr(c0 + b, b).wait()
                write_descr(c0 + b, b).start()
            for b in range(_NBUF):
                write_descr(c0 + b, b).wait()
                gather_descr(c0 + b + _NBUF, b).start()
            return carry

        lax.fori_loop(0, nouter - 1, outer, 0)

        c0 = (nouter - 1) * _NBUF
        for b in range(_NBUF):
            gather_descr(c0 + b, b).wait()
            write_descr(c0 + b, b).start()
        for b in range(_NBUF):
            write_descr(c0 + b, b).wait()

    return emb(x_flat, weights)


def kernel(x, weights):
    batch = x.size
    b_per_w = batch // _NW
    nchunk = b_per_w // _CHUNK
    x_flat = x.reshape(_NW, nchunk, _CHUNK)
    out = _emb_lookup(x_flat, weights, b_per_w, nchunk)
    return out.reshape(x.shape + (_D,))
